# baseline (speedup 1.0000x reference)
import jax
import jax.numpy as jnp
from jax.experimental import pallas as pl
from jax.experimental.pallas import tpu as pltpu

F32 = jnp.float32
HI = jax.lax.Precision.HIGHEST

B, T, HW, VDIM, ADIM, DMODEL, M = 128, 10, 49, 512, 128, 256, 1024
BT = B * T
R1 = 32
BB = 16
RB = BB * T
NEG = -1e30
EPS = 1e-05


def _softmax(x, axis):
    m = jnp.max(x, axis=axis, keepdims=True)
    e = jnp.exp(x - m)
    return e / jnp.sum(e, axis=axis, keepdims=True)


def _ln(x, g, b):
    mu = jnp.mean(x, axis=-1, keepdims=True)
    xc = x - mu
    var = jnp.mean(xc * xc, axis=-1, keepdims=True)
    return xc / jnp.sqrt(var + 1e-5) * g + b


def _dot(a, b, prec=None):
    return jax.lax.dot_general(a, b, (((1,), (0,)), ((), ())),
                               preferred_element_type=F32, precision=prec)


def _dot_t(a, b, prec=None):
    return jax.lax.dot_general(a, b, (((1,), (1,)), ((), ())),
                               preferred_element_type=F32, precision=prec)


def _full_spec(shape):
    n = len(shape)
    return pl.BlockSpec(shape, lambda *idx: (0,) * n)


def _se_body(ave_ref, sw1, sw2, out_ref):
    out_ref[...] = _dot(jax.nn.relu(_dot(ave_ref[...], sw1[...])), sw2[...])


def _kq_body(vf2_ref, ave2_ref, avw, avb, slw, slb, out_ref):
    vf2 = vf2_ref[...]
    r = vf2.shape[0]
    q = jax.nn.relu(_dot(ave2_ref[...], avw[...]) + avb[...])
    k = jax.nn.relu(_dot(vf2.reshape(r * HW, VDIM), slw[...]) + slb[...])
    out_ref[...] = k.reshape(r, HW, DMODEL) * q[:, None, :]


def _att_body(tt_ref, atw, atb, out_ref):
    tt = tt_ref[...]
    r = tt.shape[0]
    out_ref[...] = (_dot(tt.reshape(r * HW, DMODEL), atw[...])
                    .reshape(r, HW, 1) + atb[0, 0])


def _temporal_body(x_ref, win, bin_, wq, wk, wv, wo, g1, b1,
                   f1w, f1b, f2w, f2b, g2, b2, out_ref):
    x = x_ref[0]
    h = _dot(x, win[0]) + bin_[0]
    q = _dot(h, wq[0])
    k = _dot(h, wk[0])
    v = _dot(h, wv[0])
    s = _dot_t(q, k) / 16.0
    n = s.shape[0]
    ri = jax.lax.broadcasted_iota(jnp.int32, (n, n), 0) // T
    ci = jax.lax.broadcasted_iota(jnp.int32, (n, n), 1) // T
    s = jnp.where(ri == ci, s, NEG)
    attn = _softmax(s, axis=-1)
    h = _ln(h + _dot(_dot(attn, v), wo[0]), g1[0], b1[0])
    ff = _dot(jax.nn.relu(_dot(h, f1w[0]) + f1b[0]), f2w[0]) + f2b[0]
    out_ref[0] = _ln(h + ff, g2[0], b2[0])


def _vq_body(sem_ref, embh_ref, emb_ref, full_ref, qst_ref, ph_ref):
    x = sem_ref[0]
    eh = embh_ref[...]
    xe = _dot_t(x, eh)
    e2 = _dot_t(jnp.ones((1, DMODEL), F32), eh * eh, HI)
    d = e2 + jnp.sum(x * x, axis=1, keepdims=True) - 2.0 * xe
    ph = _softmax(-jnp.sqrt(jnp.maximum(d, 0.0)), axis=1)
    ri = jax.lax.broadcasted_iota(jnp.int32, (BB, RB), 0)
    ci = jax.lax.broadcasted_iota(jnp.int32, (BB, RB), 1)
    sel = (ci // T == ri).astype(F32)
    ph_ref[0] = _dot(sel, ph, HI) * (1.0 / T)
    dmin = jnp.min(d, axis=1, keepdims=True)
    ji = jax.lax.broadcasted_iota(jnp.int32, d.shape, 1)
    idx = jnp.min(jnp.where(d <= dmin, ji, M), axis=1, keepdims=True)
    oh = (ji == idx).astype(F32)
    full_ref[0] = _dot(oh, emb_ref[...], HI)
    qh = _dot(oh, eh, HI)
    qst_ref[0] = x + (qh - x)


def _lcmcm_body(ph_ref, out_ref):
    vp = ph_ref[0]
    ap = ph_ref[1]
    lv = jnp.log(vp + 1e-10)
    la = jnp.log(ap + 1e-10)
    sc = _dot_t(ap, lv, HI) + _dot_t(vp, la, HI)
    mx = jnp.max(-sc)
    es = jnp.exp(sc + mx)
    ssum = jnp.sum(es, axis=1)
    ri = jax.lax.broadcasted_iota(jnp.int32, sc.shape, 0)
    ci = jax.lax.broadcasted_iota(jnp.int32, sc.shape, 1)
    diag = jnp.sum(jnp.where(ri == ci, es, 0.0), axis=1)
    out_ref[...] = (-jnp.mean(jnp.log(diag / (ssum + EPS)))).reshape(1, 1)


def kernel(audio_feat, video_feat, epoch, params):
    p = params
    r1 = lambda a: a.reshape(1, -1)

    vf3 = video_feat.reshape(BT, HW, VDIM)
    ave = jnp.mean(vf3, axis=1)
    s_lin = pl.pallas_call(
        _se_body,
        in_specs=[_full_spec((BT, VDIM)),
                  _full_spec((VDIM, VDIM // 8)),
                  _full_spec((VDIM // 8, VDIM))],
        out_specs=_full_spec((BT, VDIM)),
        out_shape=jax.ShapeDtypeStruct((BT, VDIM), F32),
    )(ave, p['se_w1'], p['se_w2'])
    se = jax.nn.sigmoid(s_lin)
    vf2 = vf3 * se[:, None, :] + vf3
    ave2 = jnp.mean(vf2, axis=1)
    kq = pl.pallas_call(
        _kq_body,
        grid=(BT // R1,),
        in_specs=[
            pl.BlockSpec((R1, HW, VDIM), lambda i: (i, 0, 0)),
            pl.BlockSpec((R1, VDIM), lambda i: (i, 0)),
            _full_spec((VDIM, DMODEL)),
            _full_spec((1, DMODEL)),
            _full_spec((VDIM, DMODEL)),
            _full_spec((1, DMODEL)),
        ],
        out_specs=pl.BlockSpec((R1, HW, DMODEL), lambda i: (i, 0, 0)),
        out_shape=jax.ShapeDtypeStruct((BT, HW, DMODEL), F32),
        compiler_params=pltpu.CompilerParams(
            dimension_semantics=("parallel",)),
    )(vf2, ave2, p['ave_w'], r1(p['ave_b']), p['self_w'], r1(p['self_b']))
    tt = jnp.tanh(kq)
    logits = pl.pallas_call(
        _att_body,
        grid=(BT // R1,),
        in_specs=[
            pl.BlockSpec((R1, HW, DMODEL), lambda i: (i, 0, 0)),
            _full_spec((DMODEL, 1)),
            _full_spec((1, 1)),
        ],
        out_specs=pl.BlockSpec((R1, HW, 1), lambda i: (i, 0, 0)),
        out_shape=jax.ShapeDtypeStruct((BT, HW, 1), F32),
        compiler_params=pltpu.CompilerParams(
            dimension_semantics=("parallel",)),
    )(tt, p['att_w'], r1(p['att_b']))
    att = jax.nn.softmax(logits, axis=1)
    vse = jnp.sum(att * vf2, axis=1)

    a_pad = jnp.concatenate(
        [audio_feat.reshape(BT, ADIM),
         jnp.zeros((BT, VDIM - ADIM), F32)], axis=1)
    x_both = jnp.stack([vse, a_pad])
    w_in = jnp.stack([p['v_in_w'],
                      jnp.concatenate([p['a_in_w'],
                                       jnp.zeros((VDIM - ADIM, DMODEL),
                                                 F32)], axis=0)])
    st = lambda n: jnp.stack([p['v_' + n], p['a_' + n]])
    stb = lambda n: jnp.stack([r1(p['v_' + n]), r1(p['a_' + n])])

    def mspec(shape):
        n = len(shape) - 1
        return pl.BlockSpec((1,) + shape[1:], lambda i, j: (i,) + (0,) * n)

    sems = pl.pallas_call(
        _temporal_body,
        grid=(2, B // BB),
        in_specs=[
            pl.BlockSpec((1, RB, VDIM), lambda i, j: (i, j, 0)),
            mspec((2, VDIM, DMODEL)), mspec((2, 1, DMODEL)),
            mspec((2, DMODEL, DMODEL)), mspec((2, DMODEL, DMODEL)),
            mspec((2, DMODEL, DMODEL)), mspec((2, DMODEL, DMODEL)),
            mspec((2, 1, DMODEL)), mspec((2, 1, DMODEL)),
            mspec((2, DMODEL, 4 * DMODEL)), mspec((2, 1, 4 * DMODEL)),
            mspec((2, 4 * DMODEL, DMODEL)), mspec((2, 1, DMODEL)),
            mspec((2, 1, DMODEL)), mspec((2, 1, DMODEL)),
        ],
        out_specs=pl.BlockSpec((1, RB, DMODEL), lambda i, j: (i, j, 0)),
        out_shape=jax.ShapeDtypeStruct((2, BT, DMODEL), F32),
        compiler_params=pltpu.CompilerParams(
            dimension_semantics=("parallel", "parallel")),
    )(x_both, w_in, stb('in_b'), st('wq'), st('wk'), st('wv'), st('wo'),
      stb('ln1_g'), stb('ln1_b'), st('ff1_w'), stb('ff1_b'),
      st('ff2_w'), stb('ff2_b'), stb('ln2_g'), stb('ln2_b'))

    fulls, qsts, phs = pl.pallas_call(
        _vq_body,
        grid=(2, B // BB),
        in_specs=[
            pl.BlockSpec((1, RB, DMODEL), lambda i, j: (i, j, 0)),
            pl.BlockSpec((M, DMODEL), lambda i, j: (0, i)),
            pl.BlockSpec((M, 2 * DMODEL), lambda i, j: (0, 0)),
        ],
        out_specs=[
            pl.BlockSpec((1, RB, 2 * DMODEL), lambda i, j: (i, j, 0)),
            pl.BlockSpec((1, RB, DMODEL), lambda i, j: (i, j, 0)),
            pl.BlockSpec((1, BB, M), lambda i, j: (i, j, 0)),
        ],
        out_shape=[
            jax.ShapeDtypeStruct((2, BT, 2 * DMODEL), F32),
            jax.ShapeDtypeStruct((2, BT, DMODEL), F32),
            jax.ShapeDtypeStruct((2, B, M), F32),
        ],
        compiler_params=pltpu.CompilerParams(
            dimension_semantics=("parallel", "parallel")),
    )(sems, p['embedding'], p['embedding'])

    lc = pl.pallas_call(
        _lcmcm_body,
        in_specs=[_full_spec((2, B, M))],
        out_specs=_full_spec((1, 1)),
        out_shape=jax.ShapeDtypeStruct((1, 1), F32),
    )(phs)

    a_full = fulls[1].reshape(B, T, 2 * DMODEL)
    v_full = fulls[0].reshape(B, T, 2 * DMODEL)
    a_q = qsts[1].reshape(B, T, DMODEL)
    v_q = qsts[0].reshape(B, T, DMODEL)
    return (a_full, v_full, a_q, v_q, lc[0, 0])


# speedup vs baseline: 1.0127x; 1.0127x over previous
import jax
import jax.numpy as jnp
from jax.experimental import pallas as pl
from jax.experimental.pallas import tpu as pltpu

F32 = jnp.float32
HI = jax.lax.Precision.HIGHEST

B, T, HW, VDIM, ADIM, DMODEL, M = 128, 10, 49, 512, 128, 256, 1024
BT = B * T
R1 = 128
BB = 16
RB = BB * T
NEG = -1e30
EPS = 1e-05


def _softmax(x, axis):
    m = jnp.max(x, axis=axis, keepdims=True)
    e = jnp.exp(x - m)
    return e / jnp.sum(e, axis=axis, keepdims=True)


def _ln(x, g, b):
    mu = jnp.mean(x, axis=-1, keepdims=True)
    xc = x - mu
    var = jnp.mean(xc * xc, axis=-1, keepdims=True)
    return xc / jnp.sqrt(var + 1e-5) * g + b


def _dot(a, b, prec=None):
    return jax.lax.dot_general(a, b, (((1,), (0,)), ((), ())),
                               preferred_element_type=F32, precision=prec)


def _dot_t(a, b, prec=None):
    return jax.lax.dot_general(a, b, (((1,), (1,)), ((), ())),
                               preferred_element_type=F32, precision=prec)


def _full_spec(shape):
    n = len(shape)
    return pl.BlockSpec(shape, lambda *idx: (0,) * n)


def _se_body(ave_ref, sw1, sw2, out_ref):
    out_ref[...] = _dot(jax.nn.relu(_dot(ave_ref[...], sw1[...])), sw2[...])


def _kq_body(vf2_ref, ave2_ref, avw, avb, slw, slb, out_ref):
    vf2 = vf2_ref[...]
    r = vf2.shape[0]
    q = jax.nn.relu(_dot(ave2_ref[...], avw[...]) + avb[...])
    k = jax.nn.relu(_dot(vf2.reshape(r * HW, VDIM), slw[...]) + slb[...])
    out_ref[...] = k.reshape(r, HW, DMODEL) * q[:, None, :]


def _att_body(tt_ref, atw, atb, out_ref):
    tt = tt_ref[...]
    r = tt.shape[0]
    out_ref[...] = (_dot(tt.reshape(r * HW, DMODEL), atw[...])
                    .reshape(r, HW, 1) + atb[0, 0])


def _temporal_body(x_ref, win, bin_, wq, wk, wv, wo, g1, b1,
                   f1w, f1b, f2w, f2b, g2, b2, out_ref):
    x = x_ref[0]
    h = _dot(x, win[0]) + bin_[0]
    q = _dot(h, wq[0])
    k = _dot(h, wk[0])
    v = _dot(h, wv[0])
    s = _dot_t(q, k) / 16.0
    n = s.shape[0]
    ri = jax.lax.broadcasted_iota(jnp.int32, (n, n), 0) // T
    ci = jax.lax.broadcasted_iota(jnp.int32, (n, n), 1) // T
    s = jnp.where(ri == ci, s, NEG)
    attn = _softmax(s, axis=-1)
    h = _ln(h + _dot(_dot(attn, v), wo[0]), g1[0], b1[0])
    ff = _dot(jax.nn.relu(_dot(h, f1w[0]) + f1b[0]), f2w[0]) + f2b[0]
    out_ref[0] = _ln(h + ff, g2[0], b2[0])


def _vq_body(sem_ref, embh_ref, emb_ref, full_ref, qst_ref, ph_ref):
    x = sem_ref[0]
    eh = embh_ref[...]
    xe = _dot_t(x, eh)
    e2 = _dot_t(jnp.ones((1, DMODEL), F32), eh * eh, HI)
    d = e2 + jnp.sum(x * x, axis=1, keepdims=True) - 2.0 * xe
    ph = _softmax(-jnp.sqrt(jnp.maximum(d, 0.0)), axis=1)
    ri = jax.lax.broadcasted_iota(jnp.int32, (BB, RB), 0)
    ci = jax.lax.broadcasted_iota(jnp.int32, (BB, RB), 1)
    sel = (ci // T == ri).astype(F32)
    ph_ref[0] = _dot(sel, ph, HI) * (1.0 / T)
    dmin = jnp.min(d, axis=1, keepdims=True)
    ji = jax.lax.broadcasted_iota(jnp.int32, d.shape, 1)
    idx = jnp.min(jnp.where(d <= dmin, ji, M), axis=1, keepdims=True)
    oh = (ji == idx).astype(F32)
    full_ref[0] = _dot(oh, emb_ref[...], HI)
    qh = _dot(oh, eh, HI)
    qst_ref[0] = x + (qh - x)


def _lcmcm_body(ph_ref, out_ref):
    vp = ph_ref[0]
    ap = ph_ref[1]
    lv = jnp.log(vp + 1e-10)
    la = jnp.log(ap + 1e-10)
    sc = _dot_t(ap, lv, HI) + _dot_t(vp, la, HI)
    mx = jnp.max(-sc)
    es = jnp.exp(sc + mx)
    ssum = jnp.sum(es, axis=1)
    ri = jax.lax.broadcasted_iota(jnp.int32, sc.shape, 0)
    ci = jax.lax.broadcasted_iota(jnp.int32, sc.shape, 1)
    diag = jnp.sum(jnp.where(ri == ci, es, 0.0), axis=1)
    out_ref[...] = (-jnp.mean(jnp.log(diag / (ssum + EPS)))).reshape(1, 1)


def kernel(audio_feat, video_feat, epoch, params):
    p = params
    r1 = lambda a: a.reshape(1, -1)

    vf3 = video_feat.reshape(BT, HW, VDIM)
    ave = jnp.mean(vf3, axis=1)
    s_lin = pl.pallas_call(
        _se_body,
        in_specs=[_full_spec((BT, VDIM)),
                  _full_spec((VDIM, VDIM // 8)),
                  _full_spec((VDIM // 8, VDIM))],
        out_specs=_full_spec((BT, VDIM)),
        out_shape=jax.ShapeDtypeStruct((BT, VDIM), F32),
    )(ave, p['se_w1'], p['se_w2'])
    se = jax.nn.sigmoid(s_lin)
    vf2 = vf3 * se[:, None, :] + vf3
    ave2 = jnp.mean(vf2, axis=1)
    kq = pl.pallas_call(
        _kq_body,
        grid=(BT // R1,),
        in_specs=[
            pl.BlockSpec((R1, HW, VDIM), lambda i: (i, 0, 0)),
            pl.BlockSpec((R1, VDIM), lambda i: (i, 0)),
            _full_spec((VDIM, DMODEL)),
            _full_spec((1, DMODEL)),
            _full_spec((VDIM, DMODEL)),
            _full_spec((1, DMODEL)),
        ],
        out_specs=pl.BlockSpec((R1, HW, DMODEL), lambda i: (i, 0, 0)),
        out_shape=jax.ShapeDtypeStruct((BT, HW, DMODEL), F32),
        compiler_params=pltpu.CompilerParams(
            dimension_semantics=("parallel",)),
    )(vf2, ave2, p['ave_w'], r1(p['ave_b']), p['self_w'], r1(p['self_b']))
    tt = jnp.tanh(kq)
    logits = pl.pallas_call(
        _att_body,
        grid=(BT // R1,),
        in_specs=[
            pl.BlockSpec((R1, HW, DMODEL), lambda i: (i, 0, 0)),
            _full_spec((DMODEL, 1)),
            _full_spec((1, 1)),
        ],
        out_specs=pl.BlockSpec((R1, HW, 1), lambda i: (i, 0, 0)),
        out_shape=jax.ShapeDtypeStruct((BT, HW, 1), F32),
        compiler_params=pltpu.CompilerParams(
            dimension_semantics=("parallel",)),
    )(tt, p['att_w'], r1(p['att_b']))
    att = jax.nn.softmax(logits, axis=1)
    vse = jnp.sum(att * vf2, axis=1)

    a_pad = jnp.concatenate(
        [audio_feat.reshape(BT, ADIM),
         jnp.zeros((BT, VDIM - ADIM), F32)], axis=1)
    x_both = jnp.stack([vse, a_pad])
    w_in = jnp.stack([p['v_in_w'],
                      jnp.concatenate([p['a_in_w'],
                                       jnp.zeros((VDIM - ADIM, DMODEL),
                                                 F32)], axis=0)])
    st = lambda n: jnp.stack([p['v_' + n], p['a_' + n]])
    stb = lambda n: jnp.stack([r1(p['v_' + n]), r1(p['a_' + n])])

    def mspec(shape):
        n = len(shape) - 1
        return pl.BlockSpec((1,) + shape[1:], lambda i, j: (i,) + (0,) * n)

    sems = pl.pallas_call(
        _temporal_body,
        grid=(2, B // BB),
        in_specs=[
            pl.BlockSpec((1, RB, VDIM), lambda i, j: (i, j, 0)),
            mspec((2, VDIM, DMODEL)), mspec((2, 1, DMODEL)),
            mspec((2, DMODEL, DMODEL)), mspec((2, DMODEL, DMODEL)),
            mspec((2, DMODEL, DMODEL)), mspec((2, DMODEL, DMODEL)),
            mspec((2, 1, DMODEL)), mspec((2, 1, DMODEL)),
            mspec((2, DMODEL, 4 * DMODEL)), mspec((2, 1, 4 * DMODEL)),
            mspec((2, 4 * DMODEL, DMODEL)), mspec((2, 1, DMODEL)),
            mspec((2, 1, DMODEL)), mspec((2, 1, DMODEL)),
        ],
        out_specs=pl.BlockSpec((1, RB, DMODEL), lambda i, j: (i, j, 0)),
        out_shape=jax.ShapeDtypeStruct((2, BT, DMODEL), F32),
        compiler_params=pltpu.CompilerParams(
            dimension_semantics=("parallel", "parallel")),
    )(x_both, w_in, stb('in_b'), st('wq'), st('wk'), st('wv'), st('wo'),
      stb('ln1_g'), stb('ln1_b'), st('ff1_w'), stb('ff1_b'),
      st('ff2_w'), stb('ff2_b'), stb('ln2_g'), stb('ln2_b'))

    fulls, qsts, phs = pl.pallas_call(
        _vq_body,
        grid=(2, B // BB),
        in_specs=[
            pl.BlockSpec((1, RB, DMODEL), lambda i, j: (i, j, 0)),
            pl.BlockSpec((M, DMODEL), lambda i, j: (0, i)),
            pl.BlockSpec((M, 2 * DMODEL), lambda i, j: (0, 0)),
        ],
        out_specs=[
            pl.BlockSpec((1, RB, 2 * DMODEL), lambda i, j: (i, j, 0)),
            pl.BlockSpec((1, RB, DMODEL), lambda i, j: (i, j, 0)),
            pl.BlockSpec((1, BB, M), lambda i, j: (i, j, 0)),
        ],
        out_shape=[
            jax.ShapeDtypeStruct((2, BT, 2 * DMODEL), F32),
            jax.ShapeDtypeStruct((2, BT, DMODEL), F32),
            jax.ShapeDtypeStruct((2, B, M), F32),
        ],
        compiler_params=pltpu.CompilerParams(
            dimension_semantics=("parallel", "parallel")),
    )(sems, p['embedding'], p['embedding'])

    lc = pl.pallas_call(
        _lcmcm_body,
        in_specs=[_full_spec((2, B, M))],
        out_specs=_full_spec((1, 1)),
        out_shape=jax.ShapeDtypeStruct((1, 1), F32),
    )(phs)

    a_full = fulls[1].reshape(B, T, 2 * DMODEL)
    v_full = fulls[0].reshape(B, T, 2 * DMODEL)
    a_q = qsts[1].reshape(B, T, DMODEL)
    v_q = qsts[0].reshape(B, T, DMODEL)
    return (a_full, v_full, a_q, v_q, lc[0, 0])
